# baseline (speedup 1.0000x reference)
import jax
import jax.numpy as jnp
from jax.experimental import pallas as pl


def _gat_conv(x, src, dst, W, att_src, att_dst, bias, num_nodes):
    loops = jnp.arange(num_nodes, dtype=src.dtype)
    s = jnp.concatenate([src, loops])
    d = jnp.concatenate([dst, loops])
    h = x @ W
    a_src = jnp.sum(h * att_src, axis=-1)
    a_dst = jnp.sum(h * att_dst, axis=-1)
    alpha = a_src[s] + a_dst[d]
    alpha = jax.nn.leaky_relu(alpha, negative_slope=0.2)
    amax = jax.ops.segment_max(alpha, d, num_segments=num_nodes)
    amax = jnp.where(jnp.isfinite(amax), amax, 0.0)
    ex = jnp.exp(alpha - amax[d])
    denom = jax.ops.segment_sum(ex, d, num_segments=num_nodes)
    coef = ex / (denom[d] + 1e-16)
    out = jax.ops.segment_sum(h[s] * coef[:, None], d, num_segments=num_nodes)
    return out + bias


def _cross_entropy_ignore(logits, target):
    mask = (target >= 0)
    t = jnp.where(mask, target, 0)
    logp = jax.nn.log_softmax(logits, axis=-1)
    nll = -jnp.take_along_axis(logp, t[:, None], axis=1)[:, 0]
    return jnp.sum(nll * mask) / jnp.maximum(jnp.sum(mask), 1)


def kernel(x, edge_index, target, bn_gamma, bn_beta, W1, att_src1, att_dst1, b1, W2, att_src2, att_dst2, b2, pool_W, pool_b, dir_W, dir_b):
    N = x.shape[0]
    src, dst = edge_index[0], edge_index[1]
    mean = jnp.mean(x, axis=0)
    var = jnp.mean((x - mean) ** 2, axis=0)
    xn = (x - mean) / jnp.sqrt(var + 1e-5) * bn_gamma + bn_beta
    direct = xn @ dir_W + dir_b
    h1 = _gat_conv(xn, src, dst, W1, att_src1, att_dst1, b1, N)
    h2 = _gat_conv(jnp.concatenate([h1, xn], axis=-1), src, dst, W2, att_src2, att_dst2, b2, N)
    pooler = h2 @ pool_W + pool_b
    mask = (target >= 0)
    loss = _cross_entropy_ignore(pooler, target) + _cross_entropy_ignore(direct, target)
    acc = jnp.sum((jnp.argmax(pooler, axis=-1) == target) & mask) / jnp.maximum(jnp.sum(mask), 1)
    return (h2, pooler, loss, acc)


# speedup vs baseline: 4.1753x; 4.1753x over previous
import functools
import jax
import jax.numpy as jnp
from jax import lax
from jax.experimental import pallas as pl
from jax.experimental.pallas import tpu as pltpu
from jax.experimental.pallas import tpu_sc as plsc

N = 10000
E = 160000
D = 768
C = 16
K3 = 3 * D

NC = 2
NS = 16
NW = NC * NS
L = 16

R = 80
NU = N // R
WAVES = (NU + NW - 1) // NW
CAP = 2048
ECH = 4000


def _lrelu(x):
    return jnp.where(x > 0, x, 0.2 * x)


def _bn_kernel(x_ref, g_ref, b_ref, scale_ref, shift_ref, s1, s2):
    i = pl.program_id(0)

    @pl.when(i == 0)
    def _():
        s1[...] = jnp.zeros_like(s1)
        s2[...] = jnp.zeros_like(s2)

    xb = x_ref[...]
    s1[...] += jnp.sum(xb, axis=0, keepdims=True)
    s2[...] += jnp.sum(xb * xb, axis=0, keepdims=True)

    @pl.when(i == pl.num_programs(0) - 1)
    def _():
        mean = s1[...] / N
        var = s2[...] / N - mean * mean
        sc = g_ref[...][None, :] * lax.rsqrt(var + 1e-5)
        scale_ref[...] = sc
        shift_ref[...] = b_ref[...][None, :] - mean * sc


def _bn_scale_shift(x, gamma, beta):
    nb = 10
    bm = N // nb
    return pl.pallas_call(
        _bn_kernel,
        grid=(nb,),
        in_specs=[
            pl.BlockSpec((bm, K3), lambda i: (i, 0)),
            pl.BlockSpec((K3,), lambda i: (0,)),
            pl.BlockSpec((K3,), lambda i: (0,)),
        ],
        out_specs=[
            pl.BlockSpec((1, K3), lambda i: (0, 0)),
            pl.BlockSpec((1, K3), lambda i: (0, 0)),
        ],
        out_shape=[
            jax.ShapeDtypeStruct((1, K3), jnp.float32),
            jax.ShapeDtypeStruct((1, K3), jnp.float32),
        ],
        scratch_shapes=[
            pltpu.VMEM((1, K3), jnp.float32),
            pltpu.VMEM((1, K3), jnp.float32),
        ],
    )(x, gamma, beta)


def _mm_kernel(with_c, with_att, *refs):
    if with_c:
        x_ref, w_ref, sc_ref, sh_ref, b_ref, c_ref = refs[:6]
        rest = refs[6:]
    else:
        x_ref, w_ref, sc_ref, sh_ref, b_ref = refs[:5]
        rest = refs[5:]
    if with_att:
        as_ref, ad_ref, aso_ref, ado_ref, out_ref = rest
    else:
        out_ref = rest[0]

    xs = x_ref[...] * sc_ref[...]
    acc = jnp.dot(xs, w_ref[...], preferred_element_type=jnp.float32)
    acc += jnp.dot(sh_ref[...], w_ref[...], preferred_element_type=jnp.float32)
    if with_c:
        acc += c_ref[...]
    if with_att:
        aso_ref[...] = jnp.sum(acc * as_ref[...], axis=1, keepdims=True)
        ado_ref[...] = jnp.sum(acc * ad_ref[...], axis=1, keepdims=True)
    out_ref[...] = acc + b_ref[...]


def _mm(x, w, scale, shift, bias, cin=None, att=None, bm=400):
    k = x.shape[1]
    bn = w.shape[1]
    nb = N // bm
    with_c = cin is not None
    with_att = att is not None
    in_specs = [
        pl.BlockSpec((bm, k), lambda i: (i, 0)),
        pl.BlockSpec((k, bn), lambda i: (0, 0)),
        pl.BlockSpec((1, k), lambda i: (0, 0)),
        pl.BlockSpec((1, k), lambda i: (0, 0)),
        pl.BlockSpec((1, bn), lambda i: (0, 0)),
    ]
    args = [x, w, scale, shift, bias]
    if with_c:
        in_specs.append(pl.BlockSpec((bm, bn), lambda i: (i, 0)))
        args.append(cin)
    out_specs = []
    out_shape = []
    if with_att:
        in_specs += [pl.BlockSpec((1, bn), lambda i: (0, 0))] * 2
        args += [att[0], att[1]]
        out_specs += [pl.BlockSpec((bm, 1), lambda i: (i, 0))] * 2
        out_shape += [jax.ShapeDtypeStruct((N, 1), jnp.float32)] * 2
    out_specs.append(pl.BlockSpec((bm, bn), lambda i: (i, 0)))
    out_shape.append(jax.ShapeDtypeStruct((N, bn), jnp.float32))
    res = pl.pallas_call(
        functools.partial(_mm_kernel, with_c, with_att),
        grid=(nb,),
        in_specs=in_specs,
        out_specs=out_specs,
        out_shape=out_shape,
    )(*args)
    if with_att:
        a_s, a_d, out = res
        return out, a_s.reshape(N), a_d.reshape(N)
    return res[0]


def _loss_kernel(p_ref, d_ref, t_ref, loss_ref, acc_ref, s_ref):
    i = pl.program_id(0)

    @pl.when(i == 0)
    def _():
        s_ref[0] = 0.0
        s_ref[1] = 0.0
        s_ref[2] = 0.0
        s_ref[3] = 0.0

    t = t_ref[0, 0, :]
    mask = t >= 0
    tc = jnp.where(mask, t, 0)
    onehot = (tc[:, None] == lax.broadcasted_iota(jnp.int32, (1, C), 1))

    def nll_sum(logits):
        m = jnp.max(logits, axis=1, keepdims=True)
        lse = jnp.log(jnp.sum(jnp.exp(logits - m), axis=1, keepdims=True)) + m
        logp = logits - lse
        pick = jnp.sum(jnp.where(onehot, logp, 0.0), axis=1)
        return -jnp.sum(jnp.where(mask, pick, 0.0))

    p = p_ref[...]
    s_ref[0] += nll_sum(p)
    s_ref[1] += nll_sum(d_ref[...])
    pred = jnp.argmax(p, axis=1).astype(jnp.int32)
    s_ref[2] += jnp.sum(jnp.where((pred == t) & mask, 1.0, 0.0))
    s_ref[3] += jnp.sum(jnp.where(mask, 1.0, 0.0))

    @pl.when(i == pl.num_programs(0) - 1)
    def _():
        denom = jnp.maximum(s_ref[3], 1.0)
        loss_ref[...] = jnp.reshape(s_ref[0] / denom + s_ref[1] / denom, (1, 1))
        acc_ref[...] = jnp.reshape(s_ref[2] / denom, (1, 1))


def _loss_acc(pooler, direct, target):
    nb = 10
    bm = N // nb
    t3 = target.reshape(nb, 1, bm)
    loss, acc = pl.pallas_call(
        _loss_kernel,
        grid=(nb,),
        in_specs=[
            pl.BlockSpec((bm, C), lambda i: (i, 0)),
            pl.BlockSpec((bm, C), lambda i: (i, 0)),
            pl.BlockSpec((1, 1, bm), lambda i: (i, 0, 0)),
        ],
        out_specs=[
            pl.BlockSpec((1, 1), lambda i: (0, 0)),
            pl.BlockSpec((1, 1), lambda i: (0, 0)),
        ],
        out_shape=[
            jax.ShapeDtypeStruct((1, 1), jnp.float32),
            jax.ShapeDtypeStruct((1, 1), jnp.float32),
        ],
        scratch_shapes=[pltpu.SMEM((4,), jnp.float32)],
    )(pooler, direct, t3)
    return loss.reshape(()), acc.reshape(())


def _bucket_of(d):
    return jax.lax.shift_right_logical(d * 52429, 22)


def _bin_body(src_hbm, dst_hbm, bsrc_hbm, bdst_hbm, cnt_hbm,
              sstage, dstage, bufs, bufd, crow):
    wid = lax.axis_index("s") * NC + lax.axis_index("c")
    nch = E // ECH

    def chunk(ci, offs):
        pltpu.sync_copy(src_hbm.at[pl.ds(ci * ECH, ECH)], sstage)
        pltpu.sync_copy(dst_hbm.at[pl.ds(ci * ECH, ECH)], dstage)

        def vstep(k, offs):
            base = k * L
            sv = sstage[pl.ds(base, L)]
            dv = dstage[pl.ds(base, L)]
            bk = _bucket_of(dv)
            new = []
            for j in range(WAVES):
                u = wid + NW * j
                m = bk == u
                c = jnp.sum(jnp.where(m, 1, 0).astype(jnp.int32))
                base_j = j * (CAP + L)
                plsc.store_compressed(bufs.at[pl.ds(base_j + offs[j], L)], sv,
                                      mask=m)
                plsc.store_compressed(bufd.at[pl.ds(base_j + offs[j], L)], dv,
                                      mask=m)
                new.append(offs[j] + c)
            return tuple(new)

        return lax.fori_loop(0, ECH // L, vstep, offs)

    offs = lax.fori_loop(0, nch, chunk, (0, 0, 0, 0))
    for j in range(WAVES):
        u = wid + NW * j

        @pl.when(u < NU)
        def _():
            crow[...] = jnp.zeros((L,), jnp.int32) + offs[j]
            pltpu.sync_copy(bufs.at[pl.ds(j * (CAP + L), CAP)],
                            bsrc_hbm.at[pl.ds(u * CAP, CAP)])
            pltpu.sync_copy(bufd.at[pl.ds(j * (CAP + L), CAP)],
                            bdst_hbm.at[pl.ds(u * CAP, CAP)])
            pltpu.sync_copy(crow.at[pl.ds(0, 8)], cnt_hbm.at[pl.ds(u * 8, 8)])


def _bin_edges(src, dst):
    mesh = plsc.VectorSubcoreMesh(core_axis_name="c", subcore_axis_name="s",
                                  num_cores=NC, num_subcores=NS)
    f = pl.kernel(
        _bin_body,
        out_type=[
            jax.ShapeDtypeStruct((NU * CAP,), jnp.int32),
            jax.ShapeDtypeStruct((NU * CAP,), jnp.int32),
            jax.ShapeDtypeStruct((NU * 8,), jnp.int32),
        ],
        mesh=mesh,
        scratch_types=[
            pltpu.VMEM((ECH,), jnp.int32),
            pltpu.VMEM((ECH,), jnp.int32),
            pltpu.VMEM((WAVES * (CAP + L),), jnp.int32),
            pltpu.VMEM((WAVES * (CAP + L),), jnp.int32),
            pltpu.VMEM((L,), jnp.int32),
        ],
        compiler_params=pltpu.CompilerParams(needs_layout_passes=False),
    )
    return f(src, dst)


def _agg_body(h_hbm, as_hbm, ad_hbm, bsrc_hbm, bdst_hbm, cnt_hbm, out_hbm,
              asv, adv, sbuf, dbuf, exbuf, acc, rows, idx16, denom, inv,
              cself, crow, sem):
    wid = lax.axis_index("s") * NC + lax.axis_index("c")
    pltpu.sync_copy(as_hbm, asv)
    pltpu.sync_copy(ad_hbm, adv)

    for w in range(WAVES):
        u = wid + NW * w

        @pl.when(u < NU)
        def _():
            base_node = u * R
            pltpu.sync_copy(cnt_hbm.at[pl.ds(u * 8, 8)], crow.at[pl.ds(0, 8)])
            cnt = crow[pl.ds(0, L)][0]
            pltpu.sync_copy(bsrc_hbm.at[pl.ds(u * CAP, CAP)],
                            sbuf.at[pl.ds(0, CAP)])
            pltpu.sync_copy(bdst_hbm.at[pl.ds(u * CAP, CAP)],
                            dbuf.at[pl.ds(0, CAP)])
            pltpu.sync_copy(h_hbm.at[pl.ds(base_node, R)], acc)

            for j in range(R // L):
                denom[pl.ds(j * L, L)] = jnp.zeros((L,), jnp.float32)

            nv = (cnt + (L - 1)) // L

            def pass_a(i, _):
                b = i * L
                sv = sbuf[pl.ds(b, L)]
                dv = dbuf[pl.ds(b, L)]
                valid = (b + lax.iota(jnp.int32, L)) < cnt
                sv = jnp.where(valid, sv, 0)
                dv = jnp.where(valid, dv, base_node)
                dl = dv - base_node
                a = plsc.load_gather(asv, [sv]) + plsc.load_gather(adv, [dv])
                ex = jnp.where(valid, jnp.exp(_lrelu(a)), 0.0)
                sbuf[pl.ds(b, L)] = sv
                dbuf[pl.ds(b, L)] = dl
                exbuf[pl.ds(b, L)] = ex
                plsc.addupdate_scatter(denom, [dl], ex)
                return 0

            lax.fori_loop(0, nv, pass_a, 0)

            for j in range(R // L):
                sl = pl.ds(j * L, L)
                nsl = pl.ds(base_node + j * L, L)
                exs = jnp.exp(_lrelu(asv[nsl] + adv[nsl]))
                dtot = denom[sl] + exs
                iv = 1.0 / (dtot + 1e-16)
                inv[sl] = iv
                cself[sl] = exs * iv

            for j in range(R // L):
                csv = cself[pl.ds(j * L, L)]
                for k in range(L):
                    r = j * L + k
                    cs = csv[k]

                    def col(jc, _, r=r, cs=cs):
                        cl = pl.ds(jc * L, L)
                        acc[r, cl] = acc[r, cl] * cs
                        return 0

                    lax.fori_loop(0, D // L, col, 0)

            def pass_a2(i, _):
                b = i * L
                dl = dbuf[pl.ds(b, L)]
                exbuf[pl.ds(b, L)] = exbuf[pl.ds(b, L)] * plsc.load_gather(inv, [dl])
                return 0

            lax.fori_loop(0, nv, pass_a2, 0)

            def pass_b(ci, _):
                b = ci * L
                idx16[...] = sbuf[pl.ds(b, L)]
                pltpu.async_copy(h_hbm.at[idx16], rows, sem).wait()
                dlv = dbuf[pl.ds(b, L)]
                cv = exbuf[pl.ds(b, L)]
                for k in range(L):
                    dlk = dlv[k]
                    ck = cv[k]

                    def col(jc, _, k=k, dlk=dlk, ck=ck):
                        cl = pl.ds(jc * L, L)
                        plsc.addupdate(acc.at[dlk, cl], ck * rows[k, cl])
                        return 0

                    lax.fori_loop(0, D // L, col, 0)
                return 0

            lax.fori_loop(0, nv, pass_b, 0)
            pltpu.sync_copy(acc, out_hbm.at[pl.ds(base_node, R)])


def _gat_aggregate(h, a_s, a_d, bsrc, bdst, cnts):
    mesh = plsc.VectorSubcoreMesh(core_axis_name="c", subcore_axis_name="s",
                                  num_cores=NC, num_subcores=NS)
    f = pl.kernel(
        _agg_body,
        out_type=jax.ShapeDtypeStruct((N, D), jnp.float32),
        mesh=mesh,
        scratch_types=[
            pltpu.VMEM((N,), jnp.float32),
            pltpu.VMEM((N,), jnp.float32),
            pltpu.VMEM((CAP + L,), jnp.int32),
            pltpu.VMEM((CAP + L,), jnp.int32),
            pltpu.VMEM((CAP + L,), jnp.float32),
            pltpu.VMEM((R, D), jnp.float32),
            pltpu.VMEM((L, D), jnp.float32),
            pltpu.VMEM((L,), jnp.int32),
            pltpu.VMEM((R,), jnp.float32),
            pltpu.VMEM((R,), jnp.float32),
            pltpu.VMEM((R,), jnp.float32),
            pltpu.VMEM((L,), jnp.int32),
            pltpu.SemaphoreType.DMA,
        ],
        compiler_params=pltpu.CompilerParams(needs_layout_passes=False),
    )
    return f(h, a_s, a_d, bsrc, bdst, cnts)


def kernel(x, edge_index, target, bn_gamma, bn_beta, W1, att_src1, att_dst1,
           b1, W2, att_src2, att_dst2, b2, pool_W, pool_b, dir_W, dir_b):
    src = edge_index[0]
    dst = edge_index[1]

    scale, shift = _bn_scale_shift(x, bn_gamma, bn_beta)
    ones768 = jnp.ones((1, D), jnp.float32)
    zeros768 = jnp.zeros((1, D), jnp.float32)

    bsrc, bdst, cnts = _bin_edges(src, dst)

    h1h, a1s, a1d = _mm(x, W1, scale, shift, zeros768,
                        att=(att_src1.reshape(1, D), att_dst1.reshape(1, D)))
    direct = _mm(x, dir_W, scale, shift, dir_b.reshape(1, C))

    agg1 = _gat_aggregate(h1h, a1s, a1d, bsrc, bdst, cnts)

    t2 = _mm(x, W2[D:], scale, shift, zeros768)
    g, a2s, a2d = _mm(agg1, W2[:D], ones768, b1.reshape(1, D), zeros768,
                      cin=t2,
                      att=(att_src2.reshape(1, D), att_dst2.reshape(1, D)))

    agg2 = _gat_aggregate(g, a2s, a2d, bsrc, bdst, cnts)

    h2 = _mm(agg2, jnp.eye(D, dtype=jnp.float32), ones768, zeros768,
             b2.reshape(1, D))
    pooler = _mm(agg2, pool_W, ones768, b2.reshape(1, D), pool_b.reshape(1, C))

    loss, acc = _loss_acc(pooler, direct, target)
    return (h2, pooler, loss, acc)


# speedup vs baseline: 5.3037x; 1.2702x over previous
import functools
import jax
import jax.numpy as jnp
from jax import lax
from jax.experimental import pallas as pl
from jax.experimental.pallas import tpu as pltpu
from jax.experimental.pallas import tpu_sc as plsc

N = 10000
E = 160000
D = 768
C = 16
K3 = 3 * D

NC = 2
NS = 16
NW = NC * NS
L = 16

R = 80
NU = N // R
WAVES = (NU + NW - 1) // NW
CAP = 2048
ECH = 4000


def _lrelu(x):
    return jnp.where(x > 0, x, 0.2 * x)


def _bn_kernel(x_ref, g_ref, b_ref, scale_ref, shift_ref, s1, s2):
    i = pl.program_id(0)

    @pl.when(i == 0)
    def _():
        s1[...] = jnp.zeros_like(s1)
        s2[...] = jnp.zeros_like(s2)

    xb = x_ref[...]
    s1[...] += jnp.sum(xb, axis=0, keepdims=True)
    s2[...] += jnp.sum(xb * xb, axis=0, keepdims=True)

    @pl.when(i == pl.num_programs(0) - 1)
    def _():
        mean = s1[...] / N
        var = s2[...] / N - mean * mean
        sc = g_ref[...][None, :] * lax.rsqrt(var + 1e-5)
        scale_ref[...] = sc
        shift_ref[...] = b_ref[...][None, :] - mean * sc


def _bn_scale_shift(x, gamma, beta):
    nb = 10
    bm = N // nb
    return pl.pallas_call(
        _bn_kernel,
        grid=(nb,),
        in_specs=[
            pl.BlockSpec((bm, K3), lambda i: (i, 0)),
            pl.BlockSpec((K3,), lambda i: (0,)),
            pl.BlockSpec((K3,), lambda i: (0,)),
        ],
        out_specs=[
            pl.BlockSpec((1, K3), lambda i: (0, 0)),
            pl.BlockSpec((1, K3), lambda i: (0, 0)),
        ],
        out_shape=[
            jax.ShapeDtypeStruct((1, K3), jnp.float32),
            jax.ShapeDtypeStruct((1, K3), jnp.float32),
        ],
        scratch_shapes=[
            pltpu.VMEM((1, K3), jnp.float32),
            pltpu.VMEM((1, K3), jnp.float32),
        ],
    )(x, gamma, beta)


def _mm_kernel(with_c, with_att, *refs):
    if with_c:
        x_ref, w_ref, sc_ref, sh_ref, b_ref, c_ref = refs[:6]
        rest = refs[6:]
    else:
        x_ref, w_ref, sc_ref, sh_ref, b_ref = refs[:5]
        rest = refs[5:]
    if with_att:
        as_ref, ad_ref, aso_ref, ado_ref, out_ref = rest
    else:
        out_ref = rest[0]

    xs = x_ref[...] * sc_ref[...]
    acc = jnp.dot(xs, w_ref[...], preferred_element_type=jnp.float32)
    acc += jnp.dot(sh_ref[...], w_ref[...], preferred_element_type=jnp.float32)
    if with_c:
        acc += c_ref[...]
    if with_att:
        aso_ref[...] = jnp.sum(acc * as_ref[...], axis=1, keepdims=True)
        ado_ref[...] = jnp.sum(acc * ad_ref[...], axis=1, keepdims=True)
    out_ref[...] = acc + b_ref[...]


def _mm(x, w, scale, shift, bias, cin=None, att=None, bm=400):
    k = x.shape[1]
    bn = w.shape[1]
    nb = N // bm
    with_c = cin is not None
    with_att = att is not None
    in_specs = [
        pl.BlockSpec((bm, k), lambda i: (i, 0)),
        pl.BlockSpec((k, bn), lambda i: (0, 0)),
        pl.BlockSpec((1, k), lambda i: (0, 0)),
        pl.BlockSpec((1, k), lambda i: (0, 0)),
        pl.BlockSpec((1, bn), lambda i: (0, 0)),
    ]
    args = [x, w, scale, shift, bias]
    if with_c:
        in_specs.append(pl.BlockSpec((bm, bn), lambda i: (i, 0)))
        args.append(cin)
    out_specs = []
    out_shape = []
    if with_att:
        in_specs += [pl.BlockSpec((1, bn), lambda i: (0, 0))] * 2
        args += [att[0], att[1]]
        out_specs += [pl.BlockSpec((bm, 1), lambda i: (i, 0))] * 2
        out_shape += [jax.ShapeDtypeStruct((N, 1), jnp.float32)] * 2
    out_specs.append(pl.BlockSpec((bm, bn), lambda i: (i, 0)))
    out_shape.append(jax.ShapeDtypeStruct((N, bn), jnp.float32))
    res = pl.pallas_call(
        functools.partial(_mm_kernel, with_c, with_att),
        grid=(nb,),
        in_specs=in_specs,
        out_specs=out_specs,
        out_shape=out_shape,
    )(*args)
    if with_att:
        a_s, a_d, out = res
        return out, a_s.reshape(N), a_d.reshape(N)
    return res[0]


def _loss_kernel(p_ref, d_ref, t_ref, loss_ref, acc_ref, s_ref):
    i = pl.program_id(0)

    @pl.when(i == 0)
    def _():
        s_ref[0] = 0.0
        s_ref[1] = 0.0
        s_ref[2] = 0.0
        s_ref[3] = 0.0

    t = t_ref[0, 0, :]
    mask = t >= 0
    tc = jnp.where(mask, t, 0)
    onehot = (tc[:, None] == lax.broadcasted_iota(jnp.int32, (1, C), 1))

    def nll_sum(logits):
        m = jnp.max(logits, axis=1, keepdims=True)
        lse = jnp.log(jnp.sum(jnp.exp(logits - m), axis=1, keepdims=True)) + m
        logp = logits - lse
        pick = jnp.sum(jnp.where(onehot, logp, 0.0), axis=1)
        return -jnp.sum(jnp.where(mask, pick, 0.0))

    p = p_ref[...]
    s_ref[0] += nll_sum(p)
    s_ref[1] += nll_sum(d_ref[...])
    pred = jnp.argmax(p, axis=1).astype(jnp.int32)
    s_ref[2] += jnp.sum(jnp.where((pred == t) & mask, 1.0, 0.0))
    s_ref[3] += jnp.sum(jnp.where(mask, 1.0, 0.0))

    @pl.when(i == pl.num_programs(0) - 1)
    def _():
        denom = jnp.maximum(s_ref[3], 1.0)
        loss_ref[...] = jnp.reshape(s_ref[0] / denom + s_ref[1] / denom, (1, 1))
        acc_ref[...] = jnp.reshape(s_ref[2] / denom, (1, 1))


def _loss_acc(pooler, direct, target):
    nb = 10
    bm = N // nb
    t3 = target.reshape(nb, 1, bm)
    loss, acc = pl.pallas_call(
        _loss_kernel,
        grid=(nb,),
        in_specs=[
            pl.BlockSpec((bm, C), lambda i: (i, 0)),
            pl.BlockSpec((bm, C), lambda i: (i, 0)),
            pl.BlockSpec((1, 1, bm), lambda i: (i, 0, 0)),
        ],
        out_specs=[
            pl.BlockSpec((1, 1), lambda i: (0, 0)),
            pl.BlockSpec((1, 1), lambda i: (0, 0)),
        ],
        out_shape=[
            jax.ShapeDtypeStruct((1, 1), jnp.float32),
            jax.ShapeDtypeStruct((1, 1), jnp.float32),
        ],
        scratch_shapes=[pltpu.SMEM((4,), jnp.float32)],
    )(pooler, direct, t3)
    return loss.reshape(()), acc.reshape(())


def _bucket_of(d):
    return jax.lax.shift_right_logical(d * 52429, 22)


def _bin_body(src_hbm, dst_hbm, bsrc_hbm, bdst_hbm, cnt_hbm,
              sstage, dstage, bufs, bufd, crow):
    wid = lax.axis_index("s") * NC + lax.axis_index("c")
    nch = E // ECH

    def chunk(ci, offs):
        pltpu.sync_copy(src_hbm.at[pl.ds(ci * ECH, ECH)], sstage)
        pltpu.sync_copy(dst_hbm.at[pl.ds(ci * ECH, ECH)], dstage)

        def vstep(k, offs):
            base = k * L
            sv = sstage[pl.ds(base, L)]
            dv = dstage[pl.ds(base, L)]
            bk = _bucket_of(dv)
            new = []
            for j in range(WAVES):
                u = wid + NW * j
                m = bk == u
                c = jnp.sum(jnp.where(m, 1, 0).astype(jnp.int32))
                base_j = j * (CAP + L)
                plsc.store_compressed(bufs.at[pl.ds(base_j + offs[j], L)], sv,
                                      mask=m)
                plsc.store_compressed(bufd.at[pl.ds(base_j + offs[j], L)], dv,
                                      mask=m)
                new.append(offs[j] + c)
            return tuple(new)

        return lax.fori_loop(0, ECH // L, vstep, offs)

    offs = lax.fori_loop(0, nch, chunk, (0, 0, 0, 0))
    for j in range(WAVES):
        u = wid + NW * j

        @pl.when(u < NU)
        def _():
            crow[...] = jnp.zeros((L,), jnp.int32) + offs[j]
            pltpu.sync_copy(bufs.at[pl.ds(j * (CAP + L), CAP)],
                            bsrc_hbm.at[pl.ds(u * CAP, CAP)])
            pltpu.sync_copy(bufd.at[pl.ds(j * (CAP + L), CAP)],
                            bdst_hbm.at[pl.ds(u * CAP, CAP)])
            pltpu.sync_copy(crow.at[pl.ds(0, 8)], cnt_hbm.at[pl.ds(u * 8, 8)])


def _bin_edges(src, dst):
    mesh = plsc.VectorSubcoreMesh(core_axis_name="c", subcore_axis_name="s",
                                  num_cores=NC, num_subcores=NS)
    f = pl.kernel(
        _bin_body,
        out_type=[
            jax.ShapeDtypeStruct((NU * CAP,), jnp.int32),
            jax.ShapeDtypeStruct((NU * CAP,), jnp.int32),
            jax.ShapeDtypeStruct((NU * 8,), jnp.int32),
        ],
        mesh=mesh,
        scratch_types=[
            pltpu.VMEM((ECH,), jnp.int32),
            pltpu.VMEM((ECH,), jnp.int32),
            pltpu.VMEM((WAVES * (CAP + L),), jnp.int32),
            pltpu.VMEM((WAVES * (CAP + L),), jnp.int32),
            pltpu.VMEM((L,), jnp.int32),
        ],
        compiler_params=pltpu.CompilerParams(needs_layout_passes=False),
    )
    return f(src, dst)


def _agg_body(h_hbm, as_hbm, ad_hbm, bsrc_hbm, bdst_hbm, cnt_hbm, out_hbm,
              asv, adv, sbuf, dbuf, exbuf, acc, rows_a, rows_b, idx_a, idx_b,
              denom, inv, cself, crow, sem_a, sem_b):
    wid = lax.axis_index("s") * NC + lax.axis_index("c")
    pltpu.sync_copy(as_hbm, asv)
    pltpu.sync_copy(ad_hbm, adv)

    def wave(w, _):
        u = wid + NW * w

        @pl.when(u < NU)
        def _():
            base_node = u * R
            pltpu.sync_copy(cnt_hbm.at[pl.ds(u * 8, 8)], crow.at[pl.ds(0, 8)])
            cnt = crow[pl.ds(0, L)][0]
            pltpu.sync_copy(bsrc_hbm.at[pl.ds(u * CAP, CAP)],
                            sbuf.at[pl.ds(0, CAP)])
            pltpu.sync_copy(bdst_hbm.at[pl.ds(u * CAP, CAP)],
                            dbuf.at[pl.ds(0, CAP)])
            pltpu.sync_copy(h_hbm.at[pl.ds(base_node, R)], acc)

            for j in range(R // L):
                denom[pl.ds(j * L, L)] = jnp.zeros((L,), jnp.float32)

            nv = (cnt + (L - 1)) // L

            def pass_a(i, _):
                b = i * L
                sv = sbuf[pl.ds(b, L)]
                dv = dbuf[pl.ds(b, L)]
                valid = (b + lax.iota(jnp.int32, L)) < cnt
                sv = jnp.where(valid, sv, 0)
                dv = jnp.where(valid, dv, base_node)
                dl = dv - base_node
                a = plsc.load_gather(asv, [sv]) + plsc.load_gather(adv, [dv])
                ex = jnp.where(valid, jnp.exp(_lrelu(a)), 0.0)
                sbuf[pl.ds(b, L)] = sv
                dbuf[pl.ds(b, L)] = dl
                exbuf[pl.ds(b, L)] = ex
                plsc.addupdate_scatter(denom, [dl], ex)
                return 0

            lax.fori_loop(0, nv, pass_a, 0)

            for j in range(R // L):
                sl = pl.ds(j * L, L)
                nsl = pl.ds(base_node + j * L, L)
                exs = jnp.exp(_lrelu(asv[nsl] + adv[nsl]))
                dtot = denom[sl] + exs
                iv = 1.0 / (dtot + 1e-16)
                inv[sl] = iv
                cself[sl] = exs * iv

            for j in range(R // L):
                csv = cself[pl.ds(j * L, L)]
                for k in range(L):
                    r = j * L + k
                    cs = csv[k]

                    def col(jc, _, r=r, cs=cs):
                        cl = pl.ds(jc * L, L)
                        acc[r, cl] = acc[r, cl] * cs
                        return 0

                    lax.fori_loop(0, D // L, col, 0)

            def pass_a2(i, _):
                b = i * L
                dl = dbuf[pl.ds(b, L)]
                exbuf[pl.ds(b, L)] = exbuf[pl.ds(b, L)] * plsc.load_gather(inv, [dl])
                return 0

            lax.fori_loop(0, nv, pass_a2, 0)

            def launch(ci, idxr, rowsr, semr):
                idxr[...] = sbuf[pl.ds(ci * L, L)]
                pltpu.async_copy(h_hbm.at[idxr], rowsr, semr)

            def process(ci, rowsr):
                b = ci * L
                dlv = dbuf[pl.ds(b, L)]
                cv = exbuf[pl.ds(b, L)]
                for k in range(L):
                    dlk = dlv[k]
                    ck = cv[k]

                    def col(jc, _, k=k, dlk=dlk, ck=ck, rowsr=rowsr):
                        for q in range(4):
                            cl = pl.ds((jc * 4 + q) * L, L)
                            plsc.addupdate(acc.at[dlk, cl],
                                           ck * rowsr[k, cl])
                        return 0

                    lax.fori_loop(0, D // (L * 4), col, 0)

            @pl.when(nv > 0)
            def _():
                launch(0, idx_a, rows_a, sem_a)

            def pair(t, _):
                i0 = 2 * t
                i1 = i0 + 1

                @pl.when(i1 < nv)
                def _():
                    launch(i1, idx_b, rows_b, sem_b)

                pltpu.make_async_copy(h_hbm.at[idx_a], rows_a, sem_a).wait()
                process(i0, rows_a)

                @pl.when(i0 + 2 < nv)
                def _():
                    launch(i0 + 2, idx_a, rows_a, sem_a)

                @pl.when(i1 < nv)
                def _():
                    pltpu.make_async_copy(h_hbm.at[idx_b], rows_b,
                                          sem_b).wait()
                    process(i1, rows_b)

                return 0

            lax.fori_loop(0, (nv + 1) // 2, pair, 0)
            pltpu.sync_copy(acc, out_hbm.at[pl.ds(base_node, R)])

        return 0

    lax.fori_loop(0, WAVES, wave, 0)


def _gat_aggregate(h, a_s, a_d, bsrc, bdst, cnts):
    mesh = plsc.VectorSubcoreMesh(core_axis_name="c", subcore_axis_name="s",
                                  num_cores=NC, num_subcores=NS)
    f = pl.kernel(
        _agg_body,
        out_type=jax.ShapeDtypeStruct((N, D), jnp.float32),
        mesh=mesh,
        scratch_types=[
            pltpu.VMEM((N,), jnp.float32),
            pltpu.VMEM((N,), jnp.float32),
            pltpu.VMEM((CAP + L,), jnp.int32),
            pltpu.VMEM((CAP + L,), jnp.int32),
            pltpu.VMEM((CAP + L,), jnp.float32),
            pltpu.VMEM((R, D), jnp.float32),
            pltpu.VMEM((L, D), jnp.float32),
            pltpu.VMEM((L, D), jnp.float32),
            pltpu.VMEM((L,), jnp.int32),
            pltpu.VMEM((L,), jnp.int32),
            pltpu.VMEM((R,), jnp.float32),
            pltpu.VMEM((R,), jnp.float32),
            pltpu.VMEM((R,), jnp.float32),
            pltpu.VMEM((L,), jnp.int32),
            pltpu.SemaphoreType.DMA,
            pltpu.SemaphoreType.DMA,
        ],
        compiler_params=pltpu.CompilerParams(needs_layout_passes=False),
    )
    return f(h, a_s, a_d, bsrc, bdst, cnts)


def kernel(x, edge_index, target, bn_gamma, bn_beta, W1, att_src1, att_dst1,
           b1, W2, att_src2, att_dst2, b2, pool_W, pool_b, dir_W, dir_b):
    src = edge_index[0]
    dst = edge_index[1]

    scale, shift = _bn_scale_shift(x, bn_gamma, bn_beta)
    ones768 = jnp.ones((1, D), jnp.float32)
    zeros768 = jnp.zeros((1, D), jnp.float32)

    bsrc, bdst, cnts = _bin_edges(src, dst)

    h1h, a1s, a1d = _mm(x, W1, scale, shift, zeros768,
                        att=(att_src1.reshape(1, D), att_dst1.reshape(1, D)))
    direct = _mm(x, dir_W, scale, shift, dir_b.reshape(1, C))

    agg1 = _gat_aggregate(h1h, a1s, a1d, bsrc, bdst, cnts)

    t2 = _mm(x, W2[D:], scale, shift, zeros768)
    g, a2s, a2d = _mm(agg1, W2[:D], ones768, b1.reshape(1, D), zeros768,
                      cin=t2,
                      att=(att_src2.reshape(1, D), att_dst2.reshape(1, D)))

    agg2 = _gat_aggregate(g, a2s, a2d, bsrc, bdst, cnts)

    h2 = _mm(agg2, jnp.eye(D, dtype=jnp.float32), ones768, zeros768,
             b2.reshape(1, D))
    pooler = _mm(agg2, pool_W, ones768, b2.reshape(1, D), pool_b.reshape(1, C))

    loss, acc = _loss_acc(pooler, direct, target)
    return (h2, pooler, loss, acc)
